# Initial kernel scaffold; baseline (speedup 1.0000x reference)
import jax
import jax.numpy as jnp
from jax.experimental import pallas as pl


def kernel(x, rows, cols, vals, weight, bias):
    raise NotImplementedError("write your pallas kernel here")


# speedup vs baseline: 1.9410x; 1.9410x over previous
import functools

import jax
import jax.numpy as jnp
from jax import lax
from jax.experimental import pallas as pl
from jax.experimental.pallas import tpu as pltpu
from jax.experimental.pallas import tpu_sc as plsc

N = 10000
E = 320000
F = 128
K = 3
B = 4
P = 2

NC = 2
NS = 16
L = 16

C = 32
CPS = 8
SUPER = 80
EPT = SUPER * CPS * C
EPAD = NS * EPT
ZROWS = 624


def _spmm_body(xt, cols3, rows3, vals16, out,
               c0, c1, r0, r1, g0, g1, s0, s1, e0, e1,
               accum, csem0, csem1, gsem0, gsem1, ssem0, ssem1):
  core = lax.axis_index("c")
  tec = lax.axis_index("s")
  cbufs = (c0, c1)
  rbufs = (r0, r1)
  ebufs = (e0, e1)
  gbufs = (g0, g1)
  sbufs = (s0, s1)
  csems = (csem0, csem1)
  gsems = (gsem0, gsem1)
  ssems = (ssem0, ssem1)

  zvec = jnp.zeros((L,), jnp.float32)
  zbase = tec * ZROWS
  rem = ZROWS % C
  tail = N - NS * ZROWS

  def stage_fire(sup, ring):
    plane = tec * SUPER + sup
    pltpu.async_copy(cols3.at[plane], cbufs[ring], csems[ring])
    pltpu.async_copy(rows3.at[plane], rbufs[ring], csems[ring])
    pltpu.async_copy(vals16.at[tec, pl.ds(sup * CPS * C * L, CPS * C * L)],
                     ebufs[ring], csems[ring])

  def stage_wait_and_offset(sup, ring, off):
    plane = tec * SUPER + sup
    pltpu.make_async_copy(cols3.at[plane], cbufs[ring], csems[ring]).wait()
    pltpu.make_async_copy(rows3.at[plane], rbufs[ring], csems[ring]).wait()
    pltpu.make_async_copy(vals16.at[tec, pl.ds(sup * CPS * C * L, CPS * C * L)],
                          ebufs[ring], csems[ring]).wait()
    cb = cbufs[ring]

    @pl.loop(0, CPS)
    def _(q):
      for k in range(C // L):
        cb[q, pl.ds(k * L, L)] = cb[q, pl.ds(k * L, L)] + off

  def gather_fire(ring_c, q, u):
    pltpu.async_copy(xt.at[cbufs[ring_c].at[q]], gbufs[u], gsems[u])

  def gather_wait(ring_c, q, u):
    pltpu.make_async_copy(xt.at[cbufs[ring_c].at[q]], gbufs[u],
                          gsems[u]).wait()

  def scatter_fire(ring_c, q, u):
    pltpu.async_copy(sbufs[u], accum.at[rbufs[ring_c].at[q]], ssems[u],
                     add=True)

  def scatter_drain(ring_c, q, u):
    pltpu.make_async_copy(sbufs[u], accum.at[rbufs[ring_c].at[q]],
                          ssems[u]).wait()

  @pl.loop(0, B // NC)
  def _(i):
    b = core * (B // NC) + i
    off = b * N

    @pl.loop(0, C)
    def _(r):
      for k in range(F // L):
        s0[r, pl.ds(k * L, L)] = zvec

    @pl.loop(0, ZROWS // C)
    def _(q):
      pltpu.sync_copy(s0.at[pl.ds(0, C)],
                      accum.at[pl.ds(zbase + q * C, C)])
    if rem:
      pltpu.sync_copy(s0.at[pl.ds(0, rem)],
                      accum.at[pl.ds(zbase + (ZROWS // C) * C, rem)])
    if tail:
      @pl.when(tec == NS - 1)
      def _():
        pltpu.sync_copy(s0.at[pl.ds(0, tail)],
                        accum.at[pl.ds(NS * ZROWS, tail)])

    plsc.subcore_barrier()

    stage_fire(jnp.int32(0), 0)
    stage_wait_and_offset(jnp.int32(0), 0, off)
    gather_fire(0, 0, 0)
    gather_fire(0, 1, 1)

    @pl.loop(0, SUPER, step=2)
    def _(s0i):
      for ss in range(2):
        s_cur = s0i + ss
        for q in range(CPS):
          u = q % 2
          if q == 2:
            @pl.when(s_cur < SUPER - 1)
            def _():
              stage_fire(s_cur + 1, (ss + 1) % 2)
          if q == 5:
            @pl.when(s_cur < SUPER - 1)
            def _():
              stage_wait_and_offset(s_cur + 1, (ss + 1) % 2, off)

          gather_wait(ss % 2, q, u)
          if q >= 2:
            scatter_drain(ss % 2, q - 2, u)
          elif ss == 1:
            scatter_drain(0, q + 6, u)
          else:
            @pl.when(s0i > 0)
            def _():
              scatter_drain(1, q + 6, u)

          gbuf = gbufs[u]
          sbuf = sbufs[u]
          eb = ebufs[ss % 2]

          @pl.loop(0, C)
          def _(r):
            vv = eb[pl.ds((q * C + r) * L, L)]
            for k in range(F // L):
              sbuf[r, pl.ds(k * L, L)] = gbuf[r, pl.ds(k * L, L)] * vv

          scatter_fire(ss % 2, q, u)

          if q < CPS - 2:
            gather_fire(ss % 2, q + 2, u)
          else:
            @pl.when(s_cur < SUPER - 1)
            def _():
              gather_fire((ss + 1) % 2, q - (CPS - 2), u)

    scatter_drain(1, CPS - 2, 0)
    scatter_drain(1, CPS - 1, 1)

    plsc.subcore_barrier()

    obase = b * N + zbase

    @pl.loop(0, ZROWS // C)
    def _(q):
      pltpu.sync_copy(accum.at[pl.ds(zbase + q * C, C)],
                      out.at[pl.ds(obase + q * C, C)])
    if rem:
      pltpu.sync_copy(accum.at[pl.ds(zbase + (ZROWS // C) * C, rem)],
                      out.at[pl.ds(obase + (ZROWS // C) * C, rem)])
    if tail:
      @pl.when(tec == NS - 1)
      def _():
        pltpu.sync_copy(accum.at[pl.ds(NS * ZROWS, tail)],
                        out.at[pl.ds(b * N + NS * ZROWS, tail)])

    plsc.subcore_barrier()


_spmm = functools.partial(
    pl.kernel,
    out_type=jax.ShapeDtypeStruct((B * N, F), jnp.float32),
    mesh=plsc.VectorSubcoreMesh(core_axis_name="c", subcore_axis_name="s"),
    scratch_types=[
        pltpu.VMEM((CPS, C), jnp.int32),
        pltpu.VMEM((CPS, C), jnp.int32),
        pltpu.VMEM((CPS, C), jnp.int32),
        pltpu.VMEM((CPS, C), jnp.int32),
        pltpu.VMEM((C, F), jnp.float32),
        pltpu.VMEM((C, F), jnp.float32),
        pltpu.VMEM((C, F), jnp.float32),
        pltpu.VMEM((C, F), jnp.float32),
        pltpu.VMEM((CPS * C * L,), jnp.float32),
        pltpu.VMEM((CPS * C * L,), jnp.float32),
        pltpu.VMEM_SHARED((N, F), jnp.float32),
        pltpu.SemaphoreType.DMA,
        pltpu.SemaphoreType.DMA,
        pltpu.SemaphoreType.DMA,
        pltpu.SemaphoreType.DMA,
        pltpu.SemaphoreType.DMA,
        pltpu.SemaphoreType.DMA,
    ],
)(_spmm_body)


BN = 2000


def _dense_body(x0_ref, x1_ref, z_ref, w_ref, b_ref, o_ref):
  a = jnp.concatenate([x0_ref[...], x1_ref[...], z_ref[...]], axis=1)
  y = jnp.dot(a, w_ref[...], preferred_element_type=jnp.float32)
  y = jnp.maximum(y + b_ref[...], 0.0)
  o_ref[...] = jnp.max(y.reshape(BN // P, P, F), axis=1)


def _dense(x0t, x1t, zt, wcat, bias2):
  grid = (B * N // BN,)
  in_spec = pl.BlockSpec((BN, F), lambda i: (i, 0))
  return pl.pallas_call(
      _dense_body,
      grid=grid,
      in_specs=[
          in_spec, in_spec, in_spec,
          pl.BlockSpec((K * F, F), lambda i: (0, 0)),
          pl.BlockSpec((1, F), lambda i: (0, 0)),
      ],
      out_specs=pl.BlockSpec((BN // P, F), lambda i: (i, 0)),
      out_shape=jax.ShapeDtypeStruct((B * N // P, F), jnp.float32),
  )(x0t, x1t, zt, wcat, bias2)


def kernel(x, rows, cols, vals, weight, bias):
  xt = x.reshape(B * N, F)
  pad = EPAD - E
  cols2 = jnp.concatenate([cols, jnp.zeros((pad,), jnp.int32)])
  rows2 = jnp.concatenate([rows, jnp.zeros((pad,), jnp.int32)])
  vals2 = jnp.concatenate([vals, jnp.zeros((pad,), jnp.float32)])
  cols3 = cols2.reshape(NS * SUPER, CPS, C)
  rows3 = rows2.reshape(NS * SUPER, CPS, C)
  vals16 = jnp.broadcast_to(vals2[:, None], (EPAD, L)).reshape(NS, EPT * L)

  x1t = _spmm(xt, cols3, rows3, vals16)
  zt = _spmm(x1t, cols3, rows3, vals16)

  w = weight.reshape(F, K, F)
  wcat = jnp.concatenate([w[:, 0, :] - w[:, 2, :], w[:, 1, :], 2.0 * w[:, 2, :]],
                         axis=0)
  out = _dense(xt, x1t, zt, wcat, bias.reshape(1, F))
  return out.reshape(B, N // P, F)


# speedup vs baseline: 2.5977x; 1.3383x over previous
import functools

import jax
import jax.numpy as jnp
from jax import lax
from jax.experimental import pallas as pl
from jax.experimental.pallas import tpu as pltpu
from jax.experimental.pallas import tpu_sc as plsc

N = 10000
E = 320000
F = 128
K = 3
B = 4
P = 2

NC = 2
NS = 16
L = 16

C = 32
CPS = 8
SUPER = 80
EPT = SUPER * CPS * C
EPAD = NS * EPT
ZROWS = 624


def _spmm_body(xt, cols3, rows3, vals16, out,
               c0, c1, r0, r1, g0, g1, s0, s1, e0, e1,
               accum, csem0, csem1, gsem0, gsem1, ssem0, ssem1):
  core = lax.axis_index("c")
  tec = lax.axis_index("s")
  cbufs = (c0, c1)
  rbufs = (r0, r1)
  ebufs = (e0, e1)
  gbufs = (g0, g1)
  sbufs = (s0, s1)
  csems = (csem0, csem1)
  gsems = (gsem0, gsem1)
  ssems = (ssem0, ssem1)

  zvec = jnp.zeros((L,), jnp.float32)
  zbase = tec * ZROWS
  rem = ZROWS % C
  tail = N - NS * ZROWS

  def stage_fire(sup, ring):
    plane = tec * SUPER + sup
    pltpu.async_copy(cols3.at[plane], cbufs[ring], csems[ring])
    pltpu.async_copy(rows3.at[plane], rbufs[ring], csems[ring])
    pltpu.async_copy(vals16.at[tec, pl.ds(sup * CPS * C * L, CPS * C * L)],
                     ebufs[ring], csems[ring])

  def stage_wait_and_offset(sup, ring, off):
    plane = tec * SUPER + sup
    pltpu.make_async_copy(cols3.at[plane], cbufs[ring], csems[ring]).wait()
    pltpu.make_async_copy(rows3.at[plane], rbufs[ring], csems[ring]).wait()
    pltpu.make_async_copy(vals16.at[tec, pl.ds(sup * CPS * C * L, CPS * C * L)],
                          ebufs[ring], csems[ring]).wait()
    cb = cbufs[ring]

    @pl.loop(0, CPS)
    def _(q):
      for k in range(C // L):
        cb[q, pl.ds(k * L, L)] = cb[q, pl.ds(k * L, L)] + off

  def gather_fire(ring_c, q, u):
    pltpu.async_copy(xt.at[cbufs[ring_c].at[q]], gbufs[u], gsems[u])

  def gather_wait(ring_c, q, u):
    pltpu.make_async_copy(xt.at[cbufs[ring_c].at[q]], gbufs[u],
                          gsems[u]).wait()

  def scatter_fire(ring_c, q, u):
    pltpu.async_copy(sbufs[u], accum.at[rbufs[ring_c].at[q]], ssems[u],
                     add=True)

  def scatter_drain(ring_c, q, u):
    pltpu.make_async_copy(sbufs[u], accum.at[rbufs[ring_c].at[q]],
                          ssems[u]).wait()

  @pl.loop(0, B // NC)
  def _(i):
    b = core * (B // NC) + i
    off = b * N

    @pl.loop(0, C)
    def _(r):
      for k in range(F // L):
        s0[r, pl.ds(k * L, L)] = zvec

    @pl.loop(0, ZROWS // C)
    def _(q):
      pltpu.sync_copy(s0.at[pl.ds(0, C)],
                      accum.at[pl.ds(zbase + q * C, C)])
    if rem:
      pltpu.sync_copy(s0.at[pl.ds(0, rem)],
                      accum.at[pl.ds(zbase + (ZROWS // C) * C, rem)])
    if tail:
      @pl.when(tec == NS - 1)
      def _():
        pltpu.sync_copy(s0.at[pl.ds(0, tail)],
                        accum.at[pl.ds(NS * ZROWS, tail)])

    plsc.subcore_barrier()

    stage_fire(jnp.int32(0), 0)
    stage_wait_and_offset(jnp.int32(0), 0, off)
    gather_fire(0, 0, 0)
    gather_fire(0, 1, 1)

    @pl.loop(0, SUPER, step=2)
    def _(s0i):
      for ss in range(2):
        s_cur = s0i + ss
        for q in range(CPS):
          u = q % 2
          if q == 2:
            @pl.when(s_cur < SUPER - 1)
            def _():
              stage_fire(s_cur + 1, (ss + 1) % 2)
          if q == 5:
            @pl.when(s_cur < SUPER - 1)
            def _():
              stage_wait_and_offset(s_cur + 1, (ss + 1) % 2, off)

          gather_wait(ss % 2, q, u)
          if q >= 2:
            scatter_drain(ss % 2, q - 2, u)
          elif ss == 1:
            scatter_drain(0, q + 6, u)
          else:
            @pl.when(s0i > 0)
            def _():
              scatter_drain(1, q + 6, u)

          gbuf = gbufs[u]
          sbuf = sbufs[u]
          eb = ebufs[ss % 2]

          @plsc.parallel_loop(0, C, unroll=4)
          def _(r):
            vv = eb[pl.ds((q * C + r) * L, L)]
            for k in range(F // L):
              sbuf[r, pl.ds(k * L, L)] = gbuf[r, pl.ds(k * L, L)] * vv

          scatter_fire(ss % 2, q, u)

          if q < CPS - 2:
            gather_fire(ss % 2, q + 2, u)
          else:
            @pl.when(s_cur < SUPER - 1)
            def _():
              gather_fire((ss + 1) % 2, q - (CPS - 2), u)

    scatter_drain(1, CPS - 2, 0)
    scatter_drain(1, CPS - 1, 1)

    plsc.subcore_barrier()

    obase = b * N + zbase

    @pl.loop(0, ZROWS // C)
    def _(q):
      pltpu.sync_copy(accum.at[pl.ds(zbase + q * C, C)],
                      out.at[pl.ds(obase + q * C, C)])
    if rem:
      pltpu.sync_copy(accum.at[pl.ds(zbase + (ZROWS // C) * C, rem)],
                      out.at[pl.ds(obase + (ZROWS // C) * C, rem)])
    if tail:
      @pl.when(tec == NS - 1)
      def _():
        pltpu.sync_copy(accum.at[pl.ds(NS * ZROWS, tail)],
                        out.at[pl.ds(b * N + NS * ZROWS, tail)])

    plsc.subcore_barrier()


_spmm = functools.partial(
    pl.kernel,
    out_type=jax.ShapeDtypeStruct((B * N, F), jnp.float32),
    mesh=plsc.VectorSubcoreMesh(core_axis_name="c", subcore_axis_name="s"),
    scratch_types=[
        pltpu.VMEM((CPS, C), jnp.int32),
        pltpu.VMEM((CPS, C), jnp.int32),
        pltpu.VMEM((CPS, C), jnp.int32),
        pltpu.VMEM((CPS, C), jnp.int32),
        pltpu.VMEM((C, F), jnp.float32),
        pltpu.VMEM((C, F), jnp.float32),
        pltpu.VMEM((C, F), jnp.float32),
        pltpu.VMEM((C, F), jnp.float32),
        pltpu.VMEM((CPS * C * L,), jnp.float32),
        pltpu.VMEM((CPS * C * L,), jnp.float32),
        pltpu.VMEM_SHARED((N, F), jnp.float32),
        pltpu.SemaphoreType.DMA,
        pltpu.SemaphoreType.DMA,
        pltpu.SemaphoreType.DMA,
        pltpu.SemaphoreType.DMA,
        pltpu.SemaphoreType.DMA,
        pltpu.SemaphoreType.DMA,
    ],
)(_spmm_body)


BN = 2000


def _dense_body(x0_ref, x1_ref, z_ref, w_ref, b_ref, o_ref):
  a = jnp.concatenate([x0_ref[...], x1_ref[...], z_ref[...]], axis=1)
  y = jnp.dot(a, w_ref[...], preferred_element_type=jnp.float32)
  y = jnp.maximum(y + b_ref[...], 0.0)
  o_ref[...] = jnp.max(y.reshape(BN // P, P, F), axis=1)


def _dense(x0t, x1t, zt, wcat, bias2):
  grid = (B * N // BN,)
  in_spec = pl.BlockSpec((BN, F), lambda i: (i, 0))
  return pl.pallas_call(
      _dense_body,
      grid=grid,
      in_specs=[
          in_spec, in_spec, in_spec,
          pl.BlockSpec((K * F, F), lambda i: (0, 0)),
          pl.BlockSpec((1, F), lambda i: (0, 0)),
      ],
      out_specs=pl.BlockSpec((BN // P, F), lambda i: (i, 0)),
      out_shape=jax.ShapeDtypeStruct((B * N // P, F), jnp.float32),
  )(x0t, x1t, zt, wcat, bias2)


def kernel(x, rows, cols, vals, weight, bias):
  xt = x.reshape(B * N, F)
  pad = EPAD - E
  cols2 = jnp.concatenate([cols, jnp.zeros((pad,), jnp.int32)])
  rows2 = jnp.concatenate([rows, jnp.zeros((pad,), jnp.int32)])
  vals2 = jnp.concatenate([vals, jnp.zeros((pad,), jnp.float32)])
  cols3 = cols2.reshape(NS * SUPER, CPS, C)
  rows3 = rows2.reshape(NS * SUPER, CPS, C)
  vals16 = jnp.broadcast_to(vals2[:, None], (EPAD, L)).reshape(NS, EPT * L)

  x1t = _spmm(xt, cols3, rows3, vals16)
  zt = _spmm(x1t, cols3, rows3, vals16)

  w = weight.reshape(F, K, F)
  wcat = jnp.concatenate([w[:, 0, :] - w[:, 2, :], w[:, 1, :], 2.0 * w[:, 2, :]],
                         axis=0)
  out = _dense(xt, x1t, zt, wcat, bias.reshape(1, F))
  return out.reshape(B, N // P, F)


# speedup vs baseline: 2.8059x; 1.0801x over previous
import functools

import jax
import jax.numpy as jnp
from jax import lax
from jax.experimental import pallas as pl
from jax.experimental.pallas import tpu as pltpu
from jax.experimental.pallas import tpu_sc as plsc

N = 10000
E = 320000
F = 128
K = 3
B = 4
P = 2

NC = 2
NS = 16
L = 16

C = 32
CPS = 8
SUPER = 80
EPT = SUPER * CPS * C
EPAD = NS * EPT
LOOK = 4
ZROWS = 624


def _spmm_body(xt, cols3, rows3, vals16, out,
               c0, c1, r0, r1, e0, e1,
               g0, g1, g2, g3, g4, g5, g6, g7,
               accum, csem0, csem1, *sems):
  core = lax.axis_index("c")
  tec = lax.axis_index("s")
  cbufs = (c0, c1)
  rbufs = (r0, r1)
  ebufs = (e0, e1)
  gbufs = (g0, g1, g2, g3, g4, g5, g6, g7)
  csems = (csem0, csem1)
  gsems = sems[:CPS]
  ssems = sems[CPS:]

  zvec = jnp.zeros((L,), jnp.float32)
  zbase = tec * ZROWS
  rem = ZROWS % C
  tail = N - NS * ZROWS

  def stage_fire(sup, ring):
    plane = tec * SUPER + sup
    pltpu.async_copy(cols3.at[plane], cbufs[ring], csems[ring])
    pltpu.async_copy(rows3.at[plane], rbufs[ring], csems[ring])
    pltpu.async_copy(vals16.at[tec, pl.ds(sup * CPS * C * L, CPS * C * L)],
                     ebufs[ring], csems[ring])

  def stage_wait_and_offset(sup, ring, off):
    plane = tec * SUPER + sup
    pltpu.make_async_copy(cols3.at[plane], cbufs[ring], csems[ring]).wait()
    pltpu.make_async_copy(rows3.at[plane], rbufs[ring], csems[ring]).wait()
    pltpu.make_async_copy(vals16.at[tec, pl.ds(sup * CPS * C * L, CPS * C * L)],
                          ebufs[ring], csems[ring]).wait()
    cb = cbufs[ring]

    @pl.loop(0, CPS)
    def _(q):
      for k in range(C // L):
        cb[q, pl.ds(k * L, L)] = cb[q, pl.ds(k * L, L)] + off

  def gather_fire(ring_c, q, u):
    pltpu.async_copy(xt.at[cbufs[ring_c].at[q]], gbufs[u], gsems[u])

  def gather_wait(ring_c, q, u):
    pltpu.make_async_copy(xt.at[cbufs[ring_c].at[q]], gbufs[u],
                          gsems[u]).wait()

  def scatter_fire(ring_c, q, u):
    pltpu.async_copy(gbufs[u], accum.at[rbufs[ring_c].at[q]], ssems[u],
                     add=True)

  def scatter_drain(ring_c, q, u):
    pltpu.make_async_copy(gbufs[u], accum.at[rbufs[ring_c].at[q]],
                          ssems[u]).wait()

  @pl.loop(0, B // NC)
  def _(i):
    b = core * (B // NC) + i
    off = b * N

    @pl.loop(0, C)
    def _(r):
      for k in range(F // L):
        g0[r, pl.ds(k * L, L)] = zvec

    @pl.loop(0, ZROWS // C)
    def _(q):
      pltpu.sync_copy(g0.at[pl.ds(0, C)],
                      accum.at[pl.ds(zbase + q * C, C)])
    if rem:
      pltpu.sync_copy(g0.at[pl.ds(0, rem)],
                      accum.at[pl.ds(zbase + (ZROWS // C) * C, rem)])
    if tail:
      @pl.when(tec == NS - 1)
      def _():
        pltpu.sync_copy(g0.at[pl.ds(0, tail)],
                        accum.at[pl.ds(NS * ZROWS, tail)])

    plsc.subcore_barrier()

    stage_fire(jnp.int32(0), 0)
    stage_wait_and_offset(jnp.int32(0), 0, off)
    for t in range(LOOK):
      gather_fire(0, t, t)

    @pl.loop(0, SUPER, step=2)
    def _(s0i):
      for ss in range(2):
        s_cur = s0i + ss
        for q in range(CPS):
          u = q
          if q >= 2:
            scatter_drain(ss % 2, q - 2, q - 2)
          elif ss == 1:
            scatter_drain(0, q + CPS - 2, q + CPS - 2)
          else:
            @pl.when(s0i > 0)
            def _():
              scatter_drain(1, q + CPS - 2, q + CPS - 2)

          if q == 1:
            @pl.when(s_cur < SUPER - 1)
            def _():
              stage_fire(s_cur + 1, (ss + 1) % 2)
          if q == 3:
            @pl.when(s_cur < SUPER - 1)
            def _():
              stage_wait_and_offset(s_cur + 1, (ss + 1) % 2, off)

          gather_wait(ss % 2, q, u)

          gbuf = gbufs[u]
          eb = ebufs[ss % 2]

          @plsc.parallel_loop(0, C, unroll=4)
          def _(r):
            vv = eb[pl.ds((q * C + r) * L, L)]
            for k in range(F // L):
              gbuf[r, pl.ds(k * L, L)] = gbuf[r, pl.ds(k * L, L)] * vv

          scatter_fire(ss % 2, q, u)

          if q < CPS - LOOK:
            gather_fire(ss % 2, q + LOOK, q + LOOK)
          else:
            @pl.when(s_cur < SUPER - 1)
            def _():
              gather_fire((ss + 1) % 2, q - (CPS - LOOK), (q + LOOK) % CPS)

    scatter_drain(1, CPS - 2, CPS - 2)
    scatter_drain(1, CPS - 1, CPS - 1)

    plsc.subcore_barrier()

    obase = b * N + zbase

    @pl.loop(0, ZROWS // C)
    def _(q):
      pltpu.sync_copy(accum.at[pl.ds(zbase + q * C, C)],
                      out.at[pl.ds(obase + q * C, C)])
    if rem:
      pltpu.sync_copy(accum.at[pl.ds(zbase + (ZROWS // C) * C, rem)],
                      out.at[pl.ds(obase + (ZROWS // C) * C, rem)])
    if tail:
      @pl.when(tec == NS - 1)
      def _():
        pltpu.sync_copy(accum.at[pl.ds(NS * ZROWS, tail)],
                        out.at[pl.ds(b * N + NS * ZROWS, tail)])

    plsc.subcore_barrier()


_spmm = functools.partial(
    pl.kernel,
    out_type=jax.ShapeDtypeStruct((B * N, F), jnp.float32),
    mesh=plsc.VectorSubcoreMesh(core_axis_name="c", subcore_axis_name="s"),
    scratch_types=[
        pltpu.VMEM((CPS, C), jnp.int32),
        pltpu.VMEM((CPS, C), jnp.int32),
        pltpu.VMEM((CPS, C), jnp.int32),
        pltpu.VMEM((CPS, C), jnp.int32),
        pltpu.VMEM((CPS * C * L,), jnp.float32),
        pltpu.VMEM((CPS * C * L,), jnp.float32),
    ] + [pltpu.VMEM((C, F), jnp.float32)] * CPS
    + [pltpu.VMEM_SHARED((N, F), jnp.float32)]
    + [pltpu.SemaphoreType.DMA] * (2 + 2 * CPS),
)(_spmm_body)


BN = 2000


def _dense_body(x0_ref, x1_ref, z_ref, w_ref, b_ref, o_ref):
  a = jnp.concatenate([x0_ref[...], x1_ref[...], z_ref[...]], axis=1)
  y = jnp.dot(a, w_ref[...], preferred_element_type=jnp.float32)
  y = jnp.maximum(y + b_ref[...], 0.0)
  o_ref[...] = jnp.max(y.reshape(BN // P, P, F), axis=1)


def _dense(x0t, x1t, zt, wcat, bias2):
  grid = (B * N // BN,)
  in_spec = pl.BlockSpec((BN, F), lambda i: (i, 0))
  return pl.pallas_call(
      _dense_body,
      grid=grid,
      in_specs=[
          in_spec, in_spec, in_spec,
          pl.BlockSpec((K * F, F), lambda i: (0, 0)),
          pl.BlockSpec((1, F), lambda i: (0, 0)),
      ],
      out_specs=pl.BlockSpec((BN // P, F), lambda i: (i, 0)),
      out_shape=jax.ShapeDtypeStruct((B * N // P, F), jnp.float32),
  )(x0t, x1t, zt, wcat, bias2)


def kernel(x, rows, cols, vals, weight, bias):
  xt = x.reshape(B * N, F)
  pad = EPAD - E
  cols2 = jnp.concatenate([cols, jnp.zeros((pad,), jnp.int32)])
  rows2 = jnp.concatenate([rows, jnp.zeros((pad,), jnp.int32)])
  vals2 = jnp.concatenate([vals, jnp.zeros((pad,), jnp.float32)])
  cols3 = cols2.reshape(NS * SUPER, CPS, C)
  rows3 = rows2.reshape(NS * SUPER, CPS, C)
  vals16 = jnp.broadcast_to(vals2[:, None], (EPAD, L)).reshape(NS, EPT * L)

  x1t = _spmm(xt, cols3, rows3, vals16)
  zt = _spmm(x1t, cols3, rows3, vals16)

  w = weight.reshape(F, K, F)
  wcat = jnp.concatenate([w[:, 0, :] - w[:, 2, :], w[:, 1, :], 2.0 * w[:, 2, :]],
                         axis=0)
  out = _dense(xt, x1t, zt, wcat, bias.reshape(1, F))
  return out.reshape(B, N // P, F)
